# baseline (speedup 1.0000x reference)
import functools

import jax
import jax.numpy as jnp
import numpy as np
from jax.experimental import pallas as pl

_B, _T, _C, _E, _HD, _MIN_E = 4, 2048, 1024, 8, 64, 2
_BT = _B * _T
_RA = 512
_RQ = 512
_KC = 512
_NEG = np.float32(-1e9)
_HIGHEST = jax.lax.Precision.HIGHEST


def _stage_a(x_ref, sim_ref, gates_ref, p_ref, brep_ref,
             q_ref, k_ref, v_ref, w_ref):
    x = x_ref[...]
    xn = x / (jnp.sqrt(jnp.sum(x * x, axis=1, keepdims=True)) + 1e-12)
    sm = sim_ref[...]
    sn = sm / (jnp.sqrt(jnp.sum(sm * sm, axis=0, keepdims=True)) + 1e-12)
    logits = jnp.dot(xn, sn, preferred_element_type=jnp.float32)
    logits = logits - jax.nn.sigmoid(gates_ref[...])
    lt = logits.T
    gated = jnp.maximum(lt, 0.0)
    mask = (gated > 0).astype(jnp.float32)
    row = jax.lax.broadcasted_iota(jnp.int32, lt.shape, 0)
    m1 = jnp.max(lt, axis=0, keepdims=True)
    i1 = jnp.min(jnp.where(lt == m1, row, _E), axis=0, keepdims=True)
    l2 = jnp.where(row == i1, np.float32(-3e38), lt)
    m2 = jnp.max(l2, axis=0, keepdims=True)
    i2 = jnp.min(jnp.where(l2 == m2, row, _E), axis=0, keepdims=True)
    fb = ((row == i1) | (row == i2)).astype(jnp.float32)
    inactive = jnp.sum(mask, axis=0, keepdims=True) == 0
    mask = jnp.where(inactive, fb, mask)
    ml = jnp.where(mask > 0, gated, _NEG)
    p = jnp.exp(ml - jnp.max(ml, axis=0, keepdims=True))
    w = (mask * (p / jnp.sum(p, axis=0, keepdims=True))).T
    w_ref[...] = w
    hp = jnp.dot(x.astype(jnp.bfloat16), p_ref[...],
                 preferred_element_type=jnp.float32)
    w_rep = jnp.dot(w, brep_ref[...], preferred_element_type=jnp.float32)
    for out_ref, base in ((q_ref, 0), (k_ref, _E * _HD), (v_ref, 2 * _E * _HD)):
        ch = [hp[:, base + e * _HD:base + (e + 1) * _HD]
              * w_rep[:, base + e * _HD:base + (e + 1) * _HD]
              for e in range(_E)]
        while len(ch) > 1:
            ch = [ch[i] + ch[i + 1] for i in range(0, len(ch), 2)]
        out_ref[...] = ch[0].astype(jnp.bfloat16)


def _stage_b(q_ref, k_ref, v_ref, w_ref, oc_ref, brep_ref, out_ref):
    qi = pl.program_id(1)
    qb = q_ref[0]
    scale = np.float32(1.0 / np.sqrt(_HD))
    rowg = qi * _RQ + jax.lax.broadcasted_iota(jnp.int32, (_RQ, _KC), 0)
    colz = jax.lax.broadcasted_iota(jnp.int32, (_RQ, _KC), 1)

    def chunk(ki, carry):
        m, l, acc = carry
        kb = k_ref[0, pl.ds(ki * _KC, _KC), :]
        s = jax.lax.dot_general(qb, kb, (((1,), (1,)), ((), ())),
                                preferred_element_type=jnp.float32) * scale
        s = jax.lax.cond(
            ki == qi,
            lambda t: jnp.where(ki * _KC + colz <= rowg, t, np.float32(-1e30)),
            lambda t: t, s)
        m_new = jnp.maximum(m, jnp.max(s, axis=1, keepdims=True))
        r = jnp.exp(m - m_new)
        p = jnp.exp(s - m_new)
        vb = v_ref[0, pl.ds(ki * _KC, _KC), :]
        pv = jnp.dot(p.astype(jnp.bfloat16), vb,
                     preferred_element_type=jnp.float32)
        acc = acc * r + pv
        l = l * r + jnp.sum(p, axis=1, keepdims=True)
        return m_new, l, acc

    m0 = jnp.full((_RQ, 1), np.float32(-1e30))
    l0 = jnp.zeros((_RQ, 1), jnp.float32)
    a0 = jnp.zeros((_RQ, _HD), jnp.float32)
    m, l, acc = jax.lax.fori_loop(0, qi + 1, chunk, (m0, l0, a0))
    att = acc / l
    w = w_ref[0]
    w_rep = jnp.dot(w, brep_ref[...],
                    preferred_element_type=jnp.float32)
    att_rep = jnp.concatenate([att] * _E, axis=1)
    y = (att_rep * w_rep).astype(jnp.bfloat16)
    out_ref[0] = jnp.dot(y, oc_ref[...], preferred_element_type=jnp.float32)


def kernel(hidden_states, sim_matrix, gates, q_proj, k_proj, v_proj, o_proj):
    flat = hidden_states.reshape(_BT, _C)
    p_all = jnp.concatenate(
        [p.transpose(1, 0, 2).reshape(_C, _E * _HD)
         for p in (q_proj, k_proj, v_proj)], axis=1).astype(jnp.bfloat16)
    oc = o_proj.reshape(_E * _HD, _C).astype(jnp.bfloat16)
    gates2 = gates.reshape(1, _E)
    eidx = (np.arange(3 * _E * _HD) // _HD) % _E
    brep3 = jnp.asarray(np.arange(_E)[:, None] == eidx[None, :], np.float32)
    brep1 = brep3[:, :_E * _HD]

    na = _BT // _RA
    q, k, v, w = pl.pallas_call(
        _stage_a,
        grid=(na,),
        in_specs=[
            pl.BlockSpec((_RA, _C), lambda i: (i, 0)),
            pl.BlockSpec((_C, _E), lambda i: (0, 0)),
            pl.BlockSpec((1, _E), lambda i: (0, 0)),
            pl.BlockSpec((_C, 3 * _E * _HD), lambda i: (0, 0)),
            pl.BlockSpec((_E, 3 * _E * _HD), lambda i: (0, 0)),
        ],
        out_specs=[
            pl.BlockSpec((_RA, _HD), lambda i: (i, 0)),
            pl.BlockSpec((_RA, _HD), lambda i: (i, 0)),
            pl.BlockSpec((_RA, _HD), lambda i: (i, 0)),
            pl.BlockSpec((_RA, _E), lambda i: (i, 0)),
        ],
        out_shape=[
            jax.ShapeDtypeStruct((_BT, _HD), jnp.bfloat16),
            jax.ShapeDtypeStruct((_BT, _HD), jnp.bfloat16),
            jax.ShapeDtypeStruct((_BT, _HD), jnp.bfloat16),
            jax.ShapeDtypeStruct((_BT, _E), jnp.float32),
        ],
    )(flat, sim_matrix, gates2, p_all, brep3)

    qB = q.reshape(_B, _T, _HD)
    kB = k.reshape(_B, _T, _HD)
    vB = v.reshape(_B, _T, _HD)
    wB = w.reshape(_B, _T, _E)

    nq = _T // _RQ
    out = pl.pallas_call(
        _stage_b,
        grid=(_B, nq),
        in_specs=[
            pl.BlockSpec((1, _RQ, _HD), lambda b, i: (b, i, 0)),
            pl.BlockSpec((1, _T, _HD), lambda b, i: (b, 0, 0)),
            pl.BlockSpec((1, _T, _HD), lambda b, i: (b, 0, 0)),
            pl.BlockSpec((1, _RQ, _E), lambda b, i: (b, i, 0)),
            pl.BlockSpec((_E * _HD, _C), lambda b, i: (0, 0)),
            pl.BlockSpec((_E, _E * _HD), lambda b, i: (0, 0)),
        ],
        out_specs=pl.BlockSpec((1, _RQ, _C), lambda b, i: (b, i, 0)),
        out_shape=jax.ShapeDtypeStruct((_B, _T, _C), jnp.float32),
    )(qB, kB, vB, wB, oc, brep1)
    return out


# speedup vs baseline: 1.1422x; 1.1422x over previous
import functools

import jax
import jax.numpy as jnp
import numpy as np
from jax.experimental import pallas as pl

_B, _T, _C, _E, _HD, _MIN_E = 4, 2048, 1024, 8, 64, 2
_BT = _B * _T
_RA = 512
_RQ = 512
_KC = 512
_NEG = np.float32(-1e9)
_HIGHEST = jax.lax.Precision.HIGHEST


def _stage_a(x_ref, sim_ref, gates_ref, p_ref, brep_ref,
             q_ref, k_ref, v_ref, w_ref):
    x = x_ref[...]
    xn = x / (jnp.sqrt(jnp.sum(x * x, axis=1, keepdims=True)) + 1e-12)
    sm = sim_ref[...]
    sn = sm / (jnp.sqrt(jnp.sum(sm * sm, axis=0, keepdims=True)) + 1e-12)
    logits = jnp.dot(xn, sn, preferred_element_type=jnp.float32)
    logits = logits - jax.nn.sigmoid(gates_ref[...])
    lt = logits.T
    gated = jnp.maximum(lt, 0.0)
    mask = (gated > 0).astype(jnp.float32)
    row = jax.lax.broadcasted_iota(jnp.int32, lt.shape, 0)
    m1 = jnp.max(lt, axis=0, keepdims=True)
    i1 = jnp.min(jnp.where(lt == m1, row, _E), axis=0, keepdims=True)
    l2 = jnp.where(row == i1, np.float32(-3e38), lt)
    m2 = jnp.max(l2, axis=0, keepdims=True)
    i2 = jnp.min(jnp.where(l2 == m2, row, _E), axis=0, keepdims=True)
    fb = ((row == i1) | (row == i2)).astype(jnp.float32)
    inactive = jnp.sum(mask, axis=0, keepdims=True) == 0
    mask = jnp.where(inactive, fb, mask)
    ml = jnp.where(mask > 0, gated, _NEG)
    p = jnp.exp(ml - jnp.max(ml, axis=0, keepdims=True))
    w = (mask * (p / jnp.sum(p, axis=0, keepdims=True))).T
    w_ref[...] = w
    hp = jnp.dot(x.astype(jnp.bfloat16), p_ref[...],
                 preferred_element_type=jnp.float32)
    w_rep = jnp.dot(w, brep_ref[...], preferred_element_type=jnp.float32)
    for out_ref, base in ((q_ref, 0), (k_ref, _E * _HD), (v_ref, 2 * _E * _HD)):
        ch = [hp[:, base + e * _HD:base + (e + 1) * _HD]
              * w_rep[:, base + e * _HD:base + (e + 1) * _HD]
              for e in range(_E)]
        while len(ch) > 1:
            ch = [ch[i] + ch[i + 1] for i in range(0, len(ch), 2)]
        out_ref[...] = ch[0].astype(jnp.bfloat16)


def _stage_b(q_ref, k_ref, v_ref, w_ref, oc_ref, brep_ref, out_ref):
    qi = pl.program_id(1)
    qb = q_ref[0]
    scale = np.float32(1.0 / np.sqrt(_HD))
    rowg = qi * _RQ + jax.lax.broadcasted_iota(jnp.int32, (_RQ, _KC), 0)
    colz = jax.lax.broadcasted_iota(jnp.int32, (_RQ, _KC), 1)

    def chunk(ki, carry):
        m, l, acc = carry
        kb = k_ref[0, pl.ds(ki * _KC, _KC), :]
        s = jax.lax.dot_general(qb, kb, (((1,), (1,)), ((), ())),
                                preferred_element_type=jnp.float32) * scale
        s = jnp.where(ki * _KC + colz <= rowg, s, np.float32(-1e30))
        m_new = jnp.maximum(m, jnp.max(s, axis=1, keepdims=True))
        r = jnp.exp(m - m_new)
        p = jnp.exp(s - m_new)
        vb = v_ref[0, pl.ds(ki * _KC, _KC), :]
        pv = jnp.dot(p.astype(jnp.bfloat16), vb,
                     preferred_element_type=jnp.float32)
        acc = acc * r + pv
        l = l * r + jnp.sum(p, axis=1, keepdims=True)
        return m_new, l, acc

    m0 = jnp.full((_RQ, 1), np.float32(-1e30))
    l0 = jnp.zeros((_RQ, 1), jnp.float32)
    a0 = jnp.zeros((_RQ, _HD), jnp.float32)
    m, l, acc = jax.lax.fori_loop(0, qi + 1, chunk, (m0, l0, a0))
    att = acc / l
    w = w_ref[0]
    w_rep = jnp.dot(w, brep_ref[...],
                    preferred_element_type=jnp.float32)
    att_rep = jnp.concatenate([att] * _E, axis=1)
    y = (att_rep * w_rep).astype(jnp.bfloat16)
    out_ref[0] = jnp.dot(y, oc_ref[...], preferred_element_type=jnp.float32)


def kernel(hidden_states, sim_matrix, gates, q_proj, k_proj, v_proj, o_proj):
    flat = hidden_states.reshape(_BT, _C)
    p_all = jnp.concatenate(
        [p.transpose(1, 0, 2).reshape(_C, _E * _HD)
         for p in (q_proj, k_proj, v_proj)], axis=1).astype(jnp.bfloat16)
    oc = o_proj.reshape(_E * _HD, _C).astype(jnp.bfloat16)
    gates2 = gates.reshape(1, _E)
    eidx = (np.arange(3 * _E * _HD) // _HD) % _E
    brep3 = jnp.asarray(np.arange(_E)[:, None] == eidx[None, :], np.float32)
    brep1 = brep3[:, :_E * _HD]

    na = _BT // _RA
    q, k, v, w = pl.pallas_call(
        _stage_a,
        grid=(na,),
        in_specs=[
            pl.BlockSpec((_RA, _C), lambda i: (i, 0)),
            pl.BlockSpec((_C, _E), lambda i: (0, 0)),
            pl.BlockSpec((1, _E), lambda i: (0, 0)),
            pl.BlockSpec((_C, 3 * _E * _HD), lambda i: (0, 0)),
            pl.BlockSpec((_E, 3 * _E * _HD), lambda i: (0, 0)),
        ],
        out_specs=[
            pl.BlockSpec((_RA, _HD), lambda i: (i, 0)),
            pl.BlockSpec((_RA, _HD), lambda i: (i, 0)),
            pl.BlockSpec((_RA, _HD), lambda i: (i, 0)),
            pl.BlockSpec((_RA, _E), lambda i: (i, 0)),
        ],
        out_shape=[
            jax.ShapeDtypeStruct((_BT, _HD), jnp.bfloat16),
            jax.ShapeDtypeStruct((_BT, _HD), jnp.bfloat16),
            jax.ShapeDtypeStruct((_BT, _HD), jnp.bfloat16),
            jax.ShapeDtypeStruct((_BT, _E), jnp.float32),
        ],
    )(flat, sim_matrix, gates2, p_all, brep3)

    qB = q.reshape(_B, _T, _HD)
    kB = k.reshape(_B, _T, _HD)
    vB = v.reshape(_B, _T, _HD)
    wB = w.reshape(_B, _T, _E)

    nq = _T // _RQ
    out = pl.pallas_call(
        _stage_b,
        grid=(_B, nq),
        in_specs=[
            pl.BlockSpec((1, _RQ, _HD), lambda b, i: (b, i, 0)),
            pl.BlockSpec((1, _T, _HD), lambda b, i: (b, 0, 0)),
            pl.BlockSpec((1, _T, _HD), lambda b, i: (b, 0, 0)),
            pl.BlockSpec((1, _RQ, _E), lambda b, i: (b, i, 0)),
            pl.BlockSpec((_E * _HD, _C), lambda b, i: (0, 0)),
            pl.BlockSpec((_E, _E * _HD), lambda b, i: (0, 0)),
        ],
        out_specs=pl.BlockSpec((1, _RQ, _C), lambda b, i: (b, i, 0)),
        out_shape=jax.ShapeDtypeStruct((_B, _T, _C), jnp.float32),
    )(qB, kB, vB, wB, oc, brep1)
    return out


# speedup vs baseline: 1.1464x; 1.0036x over previous
import functools

import jax
import jax.numpy as jnp
import numpy as np
from jax.experimental import pallas as pl

_B, _T, _C, _E, _HD, _MIN_E = 4, 2048, 1024, 8, 64, 2
_BT = _B * _T
_RA = 512
_RQ = 512
_KC = 1024
_NEG = np.float32(-1e9)
_HIGHEST = jax.lax.Precision.HIGHEST


def _stage_a(x_ref, sim_ref, gates_ref, p_ref, brep_ref,
             q_ref, k_ref, v_ref, w_ref):
    x = x_ref[...]
    xn = x / (jnp.sqrt(jnp.sum(x * x, axis=1, keepdims=True)) + 1e-12)
    sm = sim_ref[...]
    sn = sm / (jnp.sqrt(jnp.sum(sm * sm, axis=0, keepdims=True)) + 1e-12)
    logits = jnp.dot(xn, sn, preferred_element_type=jnp.float32)
    logits = logits - jax.nn.sigmoid(gates_ref[...])
    lt = logits.T
    gated = jnp.maximum(lt, 0.0)
    mask = (gated > 0).astype(jnp.float32)
    row = jax.lax.broadcasted_iota(jnp.int32, lt.shape, 0)
    m1 = jnp.max(lt, axis=0, keepdims=True)
    i1 = jnp.min(jnp.where(lt == m1, row, _E), axis=0, keepdims=True)
    l2 = jnp.where(row == i1, np.float32(-3e38), lt)
    m2 = jnp.max(l2, axis=0, keepdims=True)
    i2 = jnp.min(jnp.where(l2 == m2, row, _E), axis=0, keepdims=True)
    fb = ((row == i1) | (row == i2)).astype(jnp.float32)
    inactive = jnp.sum(mask, axis=0, keepdims=True) == 0
    mask = jnp.where(inactive, fb, mask)
    ml = jnp.where(mask > 0, gated, _NEG)
    p = jnp.exp(ml - jnp.max(ml, axis=0, keepdims=True))
    w = (mask * (p / jnp.sum(p, axis=0, keepdims=True))).T
    w_ref[...] = w
    hp = jnp.dot(x.astype(jnp.bfloat16), p_ref[...],
                 preferred_element_type=jnp.float32)
    w_rep = jnp.dot(w, brep_ref[...], preferred_element_type=jnp.float32)
    for out_ref, base in ((q_ref, 0), (k_ref, _E * _HD), (v_ref, 2 * _E * _HD)):
        ch = [hp[:, base + e * _HD:base + (e + 1) * _HD]
              * w_rep[:, base + e * _HD:base + (e + 1) * _HD]
              for e in range(_E)]
        while len(ch) > 1:
            ch = [ch[i] + ch[i + 1] for i in range(0, len(ch), 2)]
        out_ref[...] = ch[0].astype(jnp.bfloat16)


def _stage_b(q_ref, k_ref, v_ref, w_ref, oc_ref, brep_ref, out_ref):
    qi = pl.program_id(1)
    qb = q_ref[0]
    scale = np.float32(1.0 / np.sqrt(_HD))
    rowg = qi * _RQ + jax.lax.broadcasted_iota(jnp.int32, (_RQ, _KC), 0)
    colz = jax.lax.broadcasted_iota(jnp.int32, (_RQ, _KC), 1)

    def chunk(ki, carry):
        m, l, acc = carry
        kb = k_ref[0, pl.ds(ki * _KC, _KC), :]
        s = jax.lax.dot_general(qb, kb, (((1,), (1,)), ((), ())),
                                preferred_element_type=jnp.float32) * scale
        s = jnp.where(ki * _KC + colz <= rowg, s, np.float32(-1e30))
        m_new = jnp.maximum(m, jnp.max(s, axis=1, keepdims=True))
        r = jnp.exp(m - m_new)
        p = jnp.exp(s - m_new)
        vb = v_ref[0, pl.ds(ki * _KC, _KC), :]
        pv = jnp.dot(p.astype(jnp.bfloat16), vb,
                     preferred_element_type=jnp.float32)
        acc = acc * r + pv
        l = l * r + jnp.sum(p, axis=1, keepdims=True)
        return m_new, l, acc

    m0 = jnp.full((_RQ, 1), np.float32(-1e30))
    l0 = jnp.zeros((_RQ, 1), jnp.float32)
    a0 = jnp.zeros((_RQ, _HD), jnp.float32)
    nchunks = (qi * _RQ) // _KC + 1
    m, l, acc = jax.lax.fori_loop(0, nchunks, chunk, (m0, l0, a0))
    att = acc / l
    w = w_ref[0]
    w_rep = jnp.dot(w, brep_ref[...],
                    preferred_element_type=jnp.float32)
    att_rep = jnp.concatenate([att] * _E, axis=1)
    y = (att_rep * w_rep).astype(jnp.bfloat16)
    out_ref[0] = jnp.dot(y, oc_ref[...], preferred_element_type=jnp.float32)


def kernel(hidden_states, sim_matrix, gates, q_proj, k_proj, v_proj, o_proj):
    flat = hidden_states.reshape(_BT, _C)
    p_all = jnp.concatenate(
        [p.transpose(1, 0, 2).reshape(_C, _E * _HD)
         for p in (q_proj, k_proj, v_proj)], axis=1).astype(jnp.bfloat16)
    oc = o_proj.reshape(_E * _HD, _C).astype(jnp.bfloat16)
    gates2 = gates.reshape(1, _E)
    eidx = (np.arange(3 * _E * _HD) // _HD) % _E
    brep3 = jnp.asarray(np.arange(_E)[:, None] == eidx[None, :], np.float32)
    brep1 = brep3[:, :_E * _HD]

    na = _BT // _RA
    q, k, v, w = pl.pallas_call(
        _stage_a,
        grid=(na,),
        in_specs=[
            pl.BlockSpec((_RA, _C), lambda i: (i, 0)),
            pl.BlockSpec((_C, _E), lambda i: (0, 0)),
            pl.BlockSpec((1, _E), lambda i: (0, 0)),
            pl.BlockSpec((_C, 3 * _E * _HD), lambda i: (0, 0)),
            pl.BlockSpec((_E, 3 * _E * _HD), lambda i: (0, 0)),
        ],
        out_specs=[
            pl.BlockSpec((_RA, _HD), lambda i: (i, 0)),
            pl.BlockSpec((_RA, _HD), lambda i: (i, 0)),
            pl.BlockSpec((_RA, _HD), lambda i: (i, 0)),
            pl.BlockSpec((_RA, _E), lambda i: (i, 0)),
        ],
        out_shape=[
            jax.ShapeDtypeStruct((_BT, _HD), jnp.bfloat16),
            jax.ShapeDtypeStruct((_BT, _HD), jnp.bfloat16),
            jax.ShapeDtypeStruct((_BT, _HD), jnp.bfloat16),
            jax.ShapeDtypeStruct((_BT, _E), jnp.float32),
        ],
    )(flat, sim_matrix, gates2, p_all, brep3)

    qB = q.reshape(_B, _T, _HD)
    kB = k.reshape(_B, _T, _HD)
    vB = v.reshape(_B, _T, _HD)
    wB = w.reshape(_B, _T, _E)

    nq = _T // _RQ
    out = pl.pallas_call(
        _stage_b,
        grid=(_B, nq),
        in_specs=[
            pl.BlockSpec((1, _RQ, _HD), lambda b, i: (b, i, 0)),
            pl.BlockSpec((1, _T, _HD), lambda b, i: (b, 0, 0)),
            pl.BlockSpec((1, _T, _HD), lambda b, i: (b, 0, 0)),
            pl.BlockSpec((1, _RQ, _E), lambda b, i: (b, i, 0)),
            pl.BlockSpec((_E * _HD, _C), lambda b, i: (0, 0)),
            pl.BlockSpec((_E, _E * _HD), lambda b, i: (0, 0)),
        ],
        out_specs=pl.BlockSpec((1, _RQ, _C), lambda b, i: (b, i, 0)),
        out_shape=jax.ShapeDtypeStruct((_B, _T, _C), jnp.float32),
    )(qB, kB, vB, wB, oc, brep1)
    return out


# speedup vs baseline: 1.1479x; 1.0013x over previous
import jax
import jax.numpy as jnp
import numpy as np
from jax.experimental import pallas as pl

_B, _T, _C, _E, _HD, _MIN_E = 4, 2048, 1024, 8, 64, 2
_BT = _B * _T
_RA = 512
_RQ = 512
_KC = 1024
_NEG = np.float32(-1e9)


def _stage_a(x_ref, sim_ref, gates_ref, p_ref, brep_ref,
             q_ref, k_ref, v_ref, w_ref):
    x = x_ref[...]
    xn = x / (jnp.sqrt(jnp.sum(x * x, axis=1, keepdims=True)) + 1e-12)
    sm = sim_ref[...]
    sn = sm / (jnp.sqrt(jnp.sum(sm * sm, axis=0, keepdims=True)) + 1e-12)
    logits = jnp.dot(xn, sn, preferred_element_type=jnp.float32)
    logits = logits - jax.nn.sigmoid(gates_ref[...])
    lt = logits.T
    gated = jnp.maximum(lt, 0.0)
    mask = (gated > 0).astype(jnp.float32)
    row = jax.lax.broadcasted_iota(jnp.int32, lt.shape, 0)
    m1 = jnp.max(lt, axis=0, keepdims=True)
    i1 = jnp.min(jnp.where(lt == m1, row, _E), axis=0, keepdims=True)
    l2 = jnp.where(row == i1, np.float32(-3e38), lt)
    m2 = jnp.max(l2, axis=0, keepdims=True)
    i2 = jnp.min(jnp.where(l2 == m2, row, _E), axis=0, keepdims=True)
    fb = ((row == i1) | (row == i2)).astype(jnp.float32)
    inactive = jnp.sum(mask, axis=0, keepdims=True) == 0
    mask = jnp.where(inactive, fb, mask)
    ml = jnp.where(mask > 0, gated, _NEG)
    p = jnp.exp(ml - jnp.max(ml, axis=0, keepdims=True))
    w = (mask * (p / jnp.sum(p, axis=0, keepdims=True))).T
    w_ref[...] = w
    hp = jnp.dot(x.astype(jnp.bfloat16), p_ref[...],
                 preferred_element_type=jnp.float32)
    w_rep = jnp.dot(w, brep_ref[...], preferred_element_type=jnp.float32)
    for out_ref, base in ((q_ref, 0), (k_ref, _E * _HD), (v_ref, 2 * _E * _HD)):
        ch = [hp[:, base + e * _HD:base + (e + 1) * _HD]
              * w_rep[:, base + e * _HD:base + (e + 1) * _HD]
              for e in range(_E)]
        while len(ch) > 1:
            ch = [ch[i] + ch[i + 1] for i in range(0, len(ch), 2)]
        out_ref[...] = ch[0].astype(jnp.bfloat16)


def _stage_b(q_ref, k_ref, v_ref, w_ref, oc_ref, brep_ref, out_ref):
    qi = pl.program_id(1)
    qb = q_ref[0]
    scale = np.float32(1.0 / np.sqrt(_HD))
    rowg = qi * _RQ + jax.lax.broadcasted_iota(jnp.int32, (_RQ, _KC), 0)
    colz = jax.lax.broadcasted_iota(jnp.int32, (_RQ, _KC), 1)

    def chunk(ki, carry):
        m, l, acc = carry
        kb = k_ref[0, pl.ds(ki * _KC, _KC), :]
        s = jax.lax.dot_general(qb, kb, (((1,), (1,)), ((), ())),
                                preferred_element_type=jnp.float32) * scale
        s = jnp.where(ki * _KC + colz <= rowg, s, np.float32(-1e30))
        m_new = jnp.maximum(m, jnp.max(s, axis=1, keepdims=True))
        r = jnp.exp(m - m_new)
        p = jnp.exp(s - m_new)
        vb = v_ref[0, pl.ds(ki * _KC, _KC), :]
        pv = jnp.dot(p.astype(jnp.bfloat16), vb,
                     preferred_element_type=jnp.float32)
        acc = acc * r + pv
        l = l * r + jnp.sum(p, axis=1, keepdims=True)
        return m_new, l, acc

    m0 = jnp.full((_RQ, 1), np.float32(-1e30))
    l0 = jnp.zeros((_RQ, 1), jnp.float32)
    a0 = jnp.zeros((_RQ, _HD), jnp.float32)
    nchunks = (qi * _RQ) // _KC + 1
    m, l, acc = jax.lax.fori_loop(0, nchunks, chunk, (m0, l0, a0))
    att = acc / l
    w = w_ref[0]
    w_rep = jnp.dot(w, brep_ref[...],
                    preferred_element_type=jnp.float32)
    att_rep = jnp.concatenate([att] * _E, axis=1)
    y = (att_rep * w_rep).astype(jnp.bfloat16)
    out_ref[0] = jnp.dot(y, oc_ref[...], preferred_element_type=jnp.float32)


def kernel(hidden_states, sim_matrix, gates, q_proj, k_proj, v_proj, o_proj):
    flat = hidden_states.reshape(_BT, _C)
    p_all = jnp.concatenate(
        [p.transpose(1, 0, 2).reshape(_C, _E * _HD)
         for p in (q_proj, k_proj, v_proj)], axis=1).astype(jnp.bfloat16)
    oc = o_proj.reshape(_E * _HD, _C).astype(jnp.bfloat16)
    gates2 = gates.reshape(1, _E)
    eidx = (np.arange(3 * _E * _HD) // _HD) % _E
    brep3 = jnp.asarray(np.arange(_E)[:, None] == eidx[None, :], np.float32)
    brep1 = brep3[:, :_E * _HD]

    na = _BT // _RA
    q, k, v, w = pl.pallas_call(
        _stage_a,
        grid=(na,),
        in_specs=[
            pl.BlockSpec((_RA, _C), lambda i: (i, 0)),
            pl.BlockSpec((_C, _E), lambda i: (0, 0)),
            pl.BlockSpec((1, _E), lambda i: (0, 0)),
            pl.BlockSpec((_C, 3 * _E * _HD), lambda i: (0, 0)),
            pl.BlockSpec((_E, 3 * _E * _HD), lambda i: (0, 0)),
        ],
        out_specs=[
            pl.BlockSpec((_RA, _HD), lambda i: (i, 0)),
            pl.BlockSpec((_RA, _HD), lambda i: (i, 0)),
            pl.BlockSpec((_RA, _HD), lambda i: (i, 0)),
            pl.BlockSpec((_RA, _E), lambda i: (i, 0)),
        ],
        out_shape=[
            jax.ShapeDtypeStruct((_BT, _HD), jnp.bfloat16),
            jax.ShapeDtypeStruct((_BT, _HD), jnp.bfloat16),
            jax.ShapeDtypeStruct((_BT, _HD), jnp.bfloat16),
            jax.ShapeDtypeStruct((_BT, _E), jnp.float32),
        ],
    )(flat, sim_matrix, gates2, p_all, brep3)

    qB = q.reshape(_B, _T, _HD)
    kB = k.reshape(_B, _T, _HD)
    vB = v.reshape(_B, _T, _HD)
    wB = w.reshape(_B, _T, _E)

    nq = _T // _RQ
    out = pl.pallas_call(
        _stage_b,
        grid=(_B, nq),
        in_specs=[
            pl.BlockSpec((1, _RQ, _HD), lambda b, i: (b, i, 0)),
            pl.BlockSpec((1, _T, _HD), lambda b, i: (b, 0, 0)),
            pl.BlockSpec((1, _T, _HD), lambda b, i: (b, 0, 0)),
            pl.BlockSpec((1, _RQ, _E), lambda b, i: (b, i, 0)),
            pl.BlockSpec((_E * _HD, _C), lambda b, i: (0, 0)),
            pl.BlockSpec((_E, _E * _HD), lambda b, i: (0, 0)),
        ],
        out_specs=pl.BlockSpec((1, _RQ, _C), lambda b, i: (b, i, 0)),
        out_shape=jax.ShapeDtypeStruct((_B, _T, _C), jnp.float32),
    )(qB, kB, vB, wB, oc, brep1)
    return out
